# baseline (speedup 1.0000x reference)
import functools

import jax
import jax.numpy as jnp
from jax import lax
from jax.experimental import pallas as pl
from jax.experimental.pallas import tpu as pltpu
from jax.experimental.pallas import tpu_sc as plsc

NUM_NODES = 1000000
D = 64
B = 16384
NC = 2
NS = 16
L = 16
NW = NC * NS
BPW = B // NW
CHUNK = 128
NCH = BPW // CHUNK


def _body(I_hbm, J_hbm, node_hbm, ctx_hbm, out_hbm,
          idx_i, idx_j, rows_i, rows_j, out_v, sem):
    wid = lax.axis_index("s") * NC + lax.axis_index("c")
    base = wid * BPW

    pltpu.sync_copy(I_hbm.at[wid], idx_i)
    pltpu.sync_copy(J_hbm.at[wid], idx_j)

    copies = []
    for ch in range(NCH):
        copies.append(pltpu.async_copy(
            node_hbm.at[idx_i.at[ch]],
            rows_i.at[pl.ds(ch * CHUNK, CHUNK)], sem))
        copies.append(pltpu.async_copy(
            ctx_hbm.at[idx_j.at[ch]],
            rows_j.at[pl.ds(ch * CHUNK, CHUNK)], sem))
    for cp in copies:
        cp.wait()

    lanes = lax.iota(jnp.int32, L)

    def group(g, _):
        row_ids = g * L + lanes

        def dstep(d, acc):
            col = jnp.zeros((L,), jnp.int32) + d
            vi = plsc.load_gather(rows_i, [row_ids, col])
            vj = plsc.load_gather(rows_j, [row_ids, col])
            return acc + vi * vj

        acc = lax.fori_loop(0, D, dstep, jnp.zeros((L,), jnp.float32))
        out_v[pl.ds(g * L, L)] = acc
        return _

    lax.fori_loop(0, BPW // L, group, 0)

    pltpu.sync_copy(out_v, out_hbm.at[pl.ds(base, BPW)])


@jax.jit
def _line_second(I3, J3, node_emb, context_emb):
    kern = functools.partial(
        pl.kernel,
        out_type=jax.ShapeDtypeStruct((B,), jnp.float32),
        mesh=plsc.VectorSubcoreMesh(core_axis_name="c", subcore_axis_name="s"),
        compiler_params=pltpu.CompilerParams(
            needs_layout_passes=False, use_tc_tiling_on_sc=False),
        scratch_types=[
            pltpu.VMEM((NCH, CHUNK), jnp.int32),
            pltpu.VMEM((NCH, CHUNK), jnp.int32),
            pltpu.VMEM((BPW, D), jnp.float32),
            pltpu.VMEM((BPW, D), jnp.float32),
            pltpu.VMEM((BPW,), jnp.float32),
            pltpu.SemaphoreType.DMA,
        ],
    )(_body)
    return kern(I3, J3, node_emb, context_emb)


def kernel(I, J, node_emb, context_emb):
    I3 = I.astype(jnp.int32).reshape(NW, NCH, CHUNK)
    J3 = J.astype(jnp.int32).reshape(NW, NCH, CHUNK)
    return _line_second(I3, J3, node_emb, context_emb)


# speedup vs baseline: 1.5505x; 1.5505x over previous
import functools

import jax
import jax.numpy as jnp
from jax import lax
from jax.experimental import pallas as pl
from jax.experimental.pallas import tpu as pltpu
from jax.experimental.pallas import tpu_sc as plsc

NUM_NODES = 1000000
D = 64
B = 16384
NC = 2
NS = 16
L = 16
NW = NC * NS
BPW = B // NW


def _body(I_hbm, J_hbm, node_hbm, ctx_hbm, out_hbm,
          idx_i, idx_j, rows_i, rows_j, out_v, sem):
    wid = lax.axis_index("s") * NC + lax.axis_index("c")
    base = wid * BPW

    pltpu.sync_copy(I_hbm.at[wid], idx_i)
    pltpu.sync_copy(J_hbm.at[wid], idx_j)

    lanes = lax.iota(jnp.int32, L)
    HALF = BPW // 2

    for h in range(2):
        hb = h * HALF

        def fetch(g, _):
            vi = idx_i[pl.ds(hb + g * L, L)]
            vj = idx_j[pl.ds(hb + g * L, L)]
            for k in range(L):
                pltpu.async_copy(node_hbm.at[vi[k]],
                                 rows_i.at[g * L + k], sem)
                pltpu.async_copy(ctx_hbm.at[vj[k]],
                                 rows_j.at[g * L + k], sem)
            return _
        lax.fori_loop(0, HALF // L, fetch, 0)
        pltpu.make_async_copy(node_hbm.at[pl.ds(0, HALF)], rows_i, sem).wait()
        pltpu.make_async_copy(ctx_hbm.at[pl.ds(0, HALF)], rows_j, sem).wait()

        def group(g, _):
            row_ids = g * L + lanes

            def dstep(d, acc):
                col = jnp.zeros((L,), jnp.int32) + d
                vi = plsc.load_gather(rows_i, [row_ids, col])
                vj = plsc.load_gather(rows_j, [row_ids, col])
                return acc + vi * vj

            acc = lax.fori_loop(0, D, dstep, jnp.zeros((L,), jnp.float32))
            out_v[pl.ds(hb + g * L, L)] = acc
            return _

        lax.fori_loop(0, HALF // L, group, 0)

    pltpu.sync_copy(out_v, out_hbm.at[pl.ds(base, BPW)])


@jax.jit
def _line_second(I2, J2, node_emb, context_emb):
    kern = functools.partial(
        pl.kernel,
        out_type=jax.ShapeDtypeStruct((B,), jnp.float32),
        mesh=plsc.VectorSubcoreMesh(core_axis_name="c", subcore_axis_name="s"),
        compiler_params=pltpu.CompilerParams(needs_layout_passes=False),
        scratch_types=[
            pltpu.VMEM((BPW,), jnp.int32),
            pltpu.VMEM((BPW,), jnp.int32),
            pltpu.VMEM((BPW // 2, D), jnp.float32),
            pltpu.VMEM((BPW // 2, D), jnp.float32),
            pltpu.VMEM((BPW,), jnp.float32),
            pltpu.SemaphoreType.DMA,
        ],
    )(_body)
    return kern(I2, J2, node_emb, context_emb)


def kernel(I, J, node_emb, context_emb):
    I2 = I.astype(jnp.int32).reshape(NW, BPW)
    J2 = J.astype(jnp.int32).reshape(NW, BPW)
    return _line_second(I2, J2, node_emb, context_emb)


# speedup vs baseline: 2.2236x; 1.4341x over previous
import functools

import jax
import jax.numpy as jnp
from jax import lax
from jax.experimental import pallas as pl
from jax.experimental.pallas import tpu as pltpu
from jax.experimental.pallas import tpu_sc as plsc

NUM_NODES = 1000000
D = 64
B = 16384
NC = 2
NS = 16
L = 16
NW = NC * NS
BPW = B // NW
ST = L
NSTG = BPW // ST
G = 8


def _body(I_hbm, J_hbm, node_hbm, ctx_hbm, out_hbm,
          idx_i, idx_j, bufs_i, bufs_j, out_v, sem_a, sem_b):
    wid = lax.axis_index("s") * NC + lax.axis_index("c")
    base = wid * BPW

    pltpu.sync_copy(I_hbm.at[wid], idx_i)
    pltpu.sync_copy(J_hbm.at[wid], idx_j)

    sems = [sem_a, sem_b]
    lanes = lax.iota(jnp.int32, L)
    seven = jnp.full((L,), 7, jnp.int32)

    def fire(s, par):
        gi = lax.shift_right_logical(idx_i[pl.ds(s * L, L)], 3)
        gj = lax.shift_right_logical(idx_j[pl.ds(s * L, L)], 3)
        for k in range(L):
            pltpu.async_copy(node_hbm.at[gi[k]], bufs_i.at[par, k],
                             sems[par])
            pltpu.async_copy(ctx_hbm.at[gj[k]], bufs_j.at[par, k],
                             sems[par])

    def drain(par):
        pltpu.make_async_copy(node_hbm.at[pl.ds(0, ST)], bufs_i.at[par],
                              sems[par]).wait()
        pltpu.make_async_copy(ctx_hbm.at[pl.ds(0, ST)], bufs_j.at[par],
                              sems[par]).wait()

    def compute(s, par):
        sub_i = idx_i[pl.ds(s * L, L)] & seven
        sub_j = idx_j[pl.ds(s * L, L)] & seven
        buf_i = bufs_i.at[par]
        buf_j = bufs_j.at[par]

        def dstep(d, acc):
            col = jnp.zeros((L,), jnp.int32) + d
            vi = plsc.load_gather(buf_i, [lanes, sub_i, col])
            vj = plsc.load_gather(buf_j, [lanes, sub_j, col])
            return acc + vi * vj

        acc = lax.fori_loop(0, D, dstep, jnp.zeros((L,), jnp.float32))
        out_v[pl.ds(s * L, L)] = acc

    fire(0, 0)
    for s in range(NSTG):
        par = s % 2
        if s + 1 < NSTG:
            fire(s + 1, 1 - par)
        drain(par)
        compute(s, par)

    pltpu.sync_copy(out_v, out_hbm.at[pl.ds(base, BPW)])


@jax.jit
def _line_second(I2, J2, node3, ctx3):
    kern = functools.partial(
        pl.kernel,
        out_type=jax.ShapeDtypeStruct((B,), jnp.float32),
        mesh=plsc.VectorSubcoreMesh(core_axis_name="c", subcore_axis_name="s"),
        compiler_params=pltpu.CompilerParams(needs_layout_passes=False),
        scratch_types=[
            pltpu.VMEM((BPW,), jnp.int32),
            pltpu.VMEM((BPW,), jnp.int32),
            pltpu.VMEM((2, ST, G, D), jnp.float32),
            pltpu.VMEM((2, ST, G, D), jnp.float32),
            pltpu.VMEM((BPW,), jnp.float32),
            pltpu.SemaphoreType.DMA,
            pltpu.SemaphoreType.DMA,
        ],
    )(_body)
    return kern(I2, J2, node3, ctx3)


def kernel(I, J, node_emb, context_emb):
    I2 = I.astype(jnp.int32).reshape(NW, BPW)
    J2 = J.astype(jnp.int32).reshape(NW, BPW)
    node3 = node_emb.reshape(NUM_NODES // G, G, D)
    ctx3 = context_emb.reshape(NUM_NODES // G, G, D)
    return _line_second(I2, J2, node3, ctx3)
